# baseline (speedup 1.0000x reference)
import functools

import numpy as np
import jax
import jax.numpy as jnp
from jax import lax
from jax.experimental import pallas as pl
from jax.experimental.pallas import tpu as pltpu
from jax.experimental.pallas import tpu_sc as plsc

FIELD = 26
LAT = 16
FEAT = 1040000
B = 16384
FLAT = FIELD * LAT
HPAD = 832
BS = 512
NBLK = B // BS

TOT = B * FIELD
SUB = 128
CH_ROWS = 8
CH = SUB * CH_ROWS
NW = 32
PER_W = TOT // NW
NCHUNK = PER_W // CH


def _build_src() -> np.ndarray:
    src = np.zeros((HPAD,), dtype=np.int32)
    src[:FLAT] = np.arange(FLAT)
    pos = FLAT
    for g in range(1, FIELD):
        n = FIELD - g
        for k in range(n):
            i, j = k, k + g
            p_ref = 25 * i - i * (i - 1) // 2 + (j - i - 1)
            src[pos + k] = FLAT + p_ref
        pos += ((n + 7) // 8) * 8
    assert pos == HPAD
    return src


_SRC = _build_src()


RBLK = FEAT // 128
NCB = RBLK // 8
MAIN_ROWS = NCB * 8 * 128


def _conv_body(x_ref, o_ref):
    pieces = []
    for g in range(8):
        pieces += [x_ref[0, g].T, x_ref[1, g].T]
    o_ref[...] = jnp.concatenate(pieces, axis=1)


def _convert(e4):
    return pl.pallas_call(
        _conv_body,
        grid=(NCB,),
        in_specs=[pl.BlockSpec((2, 8, 8, 128), lambda i: (0, i, 0, 0))],
        out_specs=pl.BlockSpec((128, 128), lambda i: (i, 0)),
        out_shape=jax.ShapeDtypeStruct((NCB * 128, 128), jnp.float32),
    )(e4)


def _gather(emb, idx2):
    mesh = plsc.VectorSubcoreMesh(core_axis_name="c", subcore_axis_name="s")

    @functools.partial(
        pl.kernel,
        mesh=mesh,
        compiler_params=pltpu.CompilerParams(use_tc_tiling_on_sc=False),
        out_type=jax.ShapeDtypeStruct((TOT, LAT), jnp.float32),
        scratch_types=[
            pltpu.VMEM((CH_ROWS, SUB), jnp.int32),
            pltpu.VMEM((CH, LAT), jnp.float32),
            pltpu.SemaphoreType.DMA,
        ],
    )
    def k(emb_hbm, idx_hbm, out_hbm, idx_v, rows_v, sem):
        wid = lax.axis_index("s") * 2 + lax.axis_index("c")

        def body(c, carry):
            base = wid * PER_W + c * CH
            brow = wid * (PER_W // SUB) + c * CH_ROWS
            pltpu.sync_copy(idx_hbm.at[pl.ds(brow, CH_ROWS)], idx_v)
            cps = [
                pltpu.async_copy(
                    emb_hbm.at[idx_v.at[j]],
                    rows_v.at[pl.ds(j * SUB, SUB)],
                    sem,
                )
                for j in range(CH_ROWS)
            ]
            for cp in cps:
                cp.wait()
            pltpu.sync_copy(rows_v, out_hbm.at[pl.ds(base, CH)])
            return carry

        lax.fori_loop(0, NCHUNK, body, 0)

    return k(emb, idx2)


def _tc_body(xv_ref, w1_ref, b1_ref, w2_ref, b2_ref, w3_ref, b3_ref,
             w4_ref, b4_ref, out_ref):
    xv = xv_ref[...]
    nt_raw = xv.T
    x3 = nt_raw.reshape(FIELD, LAT, BS)
    mu = jnp.mean(x3, axis=1, keepdims=True)
    xc = x3 - mu
    var = jnp.mean(xc * xc, axis=1, keepdims=True)
    n3 = xc * lax.rsqrt(var + 1e-5)
    nt = n3.reshape(FLAT, BS)

    pieces = [nt]
    for g in range(1, FIELD):
        n = FIELD - g
        prod = nt[: LAT * n, :] * nt[LAT * g: LAT * (g + n), :]
        prod = prod.reshape(n, LAT, BS).sum(axis=1)
        pad = (-n) % 8
        if pad:
            prod = jnp.concatenate(
                [prod, jnp.zeros((pad, BS), jnp.float32)], axis=0)
        pieces.append(prod)
    h = jnp.concatenate(pieces, axis=0)

    a = jnp.dot(w1_ref[...], h.astype(jnp.bfloat16),
                preferred_element_type=jnp.float32)
    a = jnp.maximum(a + b1_ref[...], 0.0).astype(jnp.bfloat16)
    a = jnp.dot(w2_ref[...], a, preferred_element_type=jnp.float32)
    a = jnp.maximum(a + b2_ref[...], 0.0).astype(jnp.bfloat16)
    a = jnp.dot(w3_ref[...], a, preferred_element_type=jnp.float32)
    a = jnp.maximum(a + b3_ref[...], 0.0).astype(jnp.bfloat16)
    s = jnp.dot(w4_ref[...], a, preferred_element_type=jnp.float32)
    out_ref[...] = (s + b4_ref[...])[0]


def kernel(x, emb, W1, b1, W2, b2, W3, b3, W4, b4):
    r = x.astype(jnp.int32).reshape(TOT)
    e4 = jnp.swapaxes(emb.T.reshape(2, 8, RBLK, 128), 1, 2)
    table = _convert(e4).reshape(MAIN_ROWS, LAT)
    g = r >> 7
    slot = (((g >> 3) << 7) + (r & 127)) * 8 + (g & 7)
    in_main = r < MAIN_ROWS
    idx2 = jnp.where(in_main, slot, 0).reshape(TOT // SUB, SUB)
    xv = _gather(table, idx2)
    tail = emb[MAIN_ROWS:, :]
    xv_tail = tail[jnp.clip(r - MAIN_ROWS, 0, FEAT - MAIN_ROWS - 1)]
    xv = jnp.where(in_main[:, None], xv, xv_tail)
    xvb = xv.reshape(B, FLAT)

    w1p = W1[jnp.asarray(_SRC), :].T.astype(jnp.bfloat16)
    w4p = jnp.pad(W4.T, ((0, 7), (0, 0))).astype(jnp.bfloat16)
    b4p = jnp.pad(b4[:, None], ((0, 7), (0, 0)))
    w2t = W2.T.astype(jnp.bfloat16)
    w3t = W3.T.astype(jnp.bfloat16)

    out = pl.pallas_call(
        _tc_body,
        grid=(NBLK,),
        in_specs=[
            pl.BlockSpec((BS, FLAT), lambda i: (i, 0)),
            pl.BlockSpec((1024, HPAD), lambda i: (0, 0)),
            pl.BlockSpec((1024, 1), lambda i: (0, 0)),
            pl.BlockSpec((512, 1024), lambda i: (0, 0)),
            pl.BlockSpec((512, 1), lambda i: (0, 0)),
            pl.BlockSpec((256, 512), lambda i: (0, 0)),
            pl.BlockSpec((256, 1), lambda i: (0, 0)),
            pl.BlockSpec((8, 256), lambda i: (0, 0)),
            pl.BlockSpec((8, 1), lambda i: (0, 0)),
        ],
        out_specs=pl.BlockSpec((BS,), lambda i: (i,)),
        out_shape=jax.ShapeDtypeStruct((B,), jnp.float32),
    )(xvb, w1p, b1[:, None], w2t, b2[:, None], w3t, b3[:, None], w4p, b4p)
    return out


# speedup vs baseline: 4.1583x; 4.1583x over previous
import functools

import numpy as np
import jax
import jax.numpy as jnp
from jax import lax
from jax.experimental import pallas as pl
from jax.experimental.pallas import tpu as pltpu
from jax.experimental.pallas import tpu_sc as plsc

FIELD = 26
LAT = 16
B = 16384
FLAT = FIELD * LAT
HPAD = 832
BS = 512
NBLK = B // BS

TOT = B * FIELD
SUB = 128
CH_ROWS = 8
CH = SUB * CH_ROWS
NW = 32
PER_W = TOT // NW
NCHUNK = PER_W // CH


def _build_src() -> np.ndarray:
    src = np.zeros((HPAD,), dtype=np.int32)
    src[:FLAT] = np.arange(FLAT)
    pos = FLAT
    for g in range(1, FIELD):
        n = FIELD - g
        for k in range(n):
            i, j = k, k + g
            p_ref = 25 * i - i * (i - 1) // 2 + (j - i - 1)
            src[pos + k] = FLAT + p_ref
        pos += ((n + 7) // 8) * 8
    assert pos == HPAD
    return src


_SRC = _build_src()


def _gather(emb, idx2):
    mesh = plsc.VectorSubcoreMesh(core_axis_name="c", subcore_axis_name="s")

    @functools.partial(
        pl.kernel,
        mesh=mesh,
        compiler_params=pltpu.CompilerParams(use_tc_tiling_on_sc=False),
        out_type=jax.ShapeDtypeStruct((TOT, LAT), jnp.float32),
        scratch_types=[
            pltpu.VMEM((CH_ROWS, SUB), jnp.int32),
            pltpu.VMEM((CH, LAT), jnp.float32),
            pltpu.SemaphoreType.DMA,
        ],
    )
    def k(emb_hbm, idx_hbm, out_hbm, idx_v, rows_v, sem):
        wid = lax.axis_index("s") * 2 + lax.axis_index("c")

        def body(c, carry):
            base = wid * PER_W + c * CH
            brow = wid * (PER_W // SUB) + c * CH_ROWS
            pltpu.sync_copy(idx_hbm.at[pl.ds(brow, CH_ROWS)], idx_v)
            cps = [
                pltpu.async_copy(
                    emb_hbm.at[idx_v.at[j]],
                    rows_v.at[pl.ds(j * SUB, SUB)],
                    sem,
                )
                for j in range(CH_ROWS)
            ]
            for cp in cps:
                cp.wait()
            pltpu.sync_copy(rows_v, out_hbm.at[pl.ds(base, CH)])
            return carry

        lax.fori_loop(0, NCHUNK, body, 0)

    return k(emb, idx2)


def _tc_body(xv_ref, w1_ref, b1_ref, w2_ref, b2_ref, w3_ref, b3_ref,
             w4_ref, b4_ref, out_ref):
    xv = xv_ref[...]
    nt_raw = xv.T
    x3 = nt_raw.reshape(FIELD, LAT, BS)
    mu = jnp.mean(x3, axis=1, keepdims=True)
    xc = x3 - mu
    var = jnp.mean(xc * xc, axis=1, keepdims=True)
    n3 = xc * lax.rsqrt(var + 1e-5)
    nt = n3.reshape(FLAT, BS)

    pieces = [nt]
    for g in range(1, FIELD):
        n = FIELD - g
        prod = nt[: LAT * n, :] * nt[LAT * g: LAT * (g + n), :]
        prod = prod.reshape(n, LAT, BS).sum(axis=1)
        pad = (-n) % 8
        if pad:
            prod = jnp.concatenate(
                [prod, jnp.zeros((pad, BS), jnp.float32)], axis=0)
        pieces.append(prod)
    h = jnp.concatenate(pieces, axis=0)

    a = jnp.dot(w1_ref[...], h.astype(jnp.bfloat16),
                preferred_element_type=jnp.float32)
    a = jnp.maximum(a + b1_ref[...], 0.0).astype(jnp.bfloat16)
    a = jnp.dot(w2_ref[...], a, preferred_element_type=jnp.float32)
    a = jnp.maximum(a + b2_ref[...], 0.0).astype(jnp.bfloat16)
    a = jnp.dot(w3_ref[...], a, preferred_element_type=jnp.float32)
    a = jnp.maximum(a + b3_ref[...], 0.0).astype(jnp.bfloat16)
    s = jnp.dot(w4_ref[...], a, preferred_element_type=jnp.float32)
    out_ref[...] = (s + b4_ref[...])[0]


def kernel(x, emb, W1, b1, W2, b2, W3, b3, W4, b4):
    idx2 = x.astype(jnp.int32).reshape(TOT // SUB, SUB)
    xv = _gather(emb, idx2)
    xvb = xv.reshape(B, FLAT)

    w1p = W1[jnp.asarray(_SRC), :].T.astype(jnp.bfloat16)
    w4p = jnp.pad(W4.T, ((0, 7), (0, 0))).astype(jnp.bfloat16)
    b4p = jnp.pad(b4[:, None], ((0, 7), (0, 0)))
    w2t = W2.T.astype(jnp.bfloat16)
    w3t = W3.T.astype(jnp.bfloat16)

    out = pl.pallas_call(
        _tc_body,
        grid=(NBLK,),
        in_specs=[
            pl.BlockSpec((BS, FLAT), lambda i: (i, 0)),
            pl.BlockSpec((1024, HPAD), lambda i: (0, 0)),
            pl.BlockSpec((1024, 1), lambda i: (0, 0)),
            pl.BlockSpec((512, 1024), lambda i: (0, 0)),
            pl.BlockSpec((512, 1), lambda i: (0, 0)),
            pl.BlockSpec((256, 512), lambda i: (0, 0)),
            pl.BlockSpec((256, 1), lambda i: (0, 0)),
            pl.BlockSpec((8, 256), lambda i: (0, 0)),
            pl.BlockSpec((8, 1), lambda i: (0, 0)),
        ],
        out_specs=pl.BlockSpec((BS,), lambda i: (i,)),
        out_shape=jax.ShapeDtypeStruct((B,), jnp.float32),
    )(xvb, w1p, b1[:, None], w2t, b2[:, None], w3t, b3[:, None], w4p, b4p)
    return out
